# baseline (speedup 1.0000x reference)
import jax
import jax.numpy as jnp
from jax.experimental import pallas as pl
from jax.experimental.pallas import tpu as pltpu

TOP_K = 8
NUM_EXPERTS = 64
HIDDEN_DIM = 4096
TOKEN_BLOCK = 512


def _route_block(x, w, logits_ref, topv_ref, topi_ref):
    m_blk = x.shape[0]
    lt = jax.lax.dot_general(
        w, x, (((1,), (1,)), ((), ())), preferred_element_type=jnp.float32
    )
    r = jax.lax.broadcasted_iota(jnp.int32, (NUM_EXPERTS, NUM_EXPERTS), 0)
    c = jax.lax.broadcasted_iota(jnp.int32, (NUM_EXPERTS, NUM_EXPERTS), 1)
    eye = (r == c).astype(jnp.float32)
    logits_ref[...] = jax.lax.dot_general(
        lt, eye, (((0,), (0,)), ((), ())), preferred_element_type=jnp.float32
    )

    eiota = jax.lax.broadcasted_iota(jnp.int32, (NUM_EXPERTS, m_blk), 0)
    work = lt
    vals, idxs = [], []
    for _ in range(TOP_K):
        m = jnp.max(work, axis=0, keepdims=True)
        idx = jnp.min(
            jnp.where(work == m, eiota, NUM_EXPERTS), axis=0, keepdims=True
        )
        vals.append(m)
        idxs.append(idx)
        work = jnp.where(eiota == idx, -jnp.inf, work)
    topv = jnp.concatenate(vals, axis=0)
    topi = jnp.concatenate(idxs, axis=0)

    e = jnp.exp(topv - topv[0:1, :])
    topv_ref[...] = e / jnp.sum(e, axis=0, keepdims=True)
    topi_ref[...] = topi


def _router_block(hsa_ref, hsb_ref, w_ref, la_ref, lb_ref, va_ref, vb_ref,
                  ia_ref, ib_ref):
    w = w_ref[...]
    _route_block(hsa_ref[...], w, la_ref, va_ref, ia_ref)
    _route_block(hsb_ref[...], w, lb_ref, vb_ref, ib_ref)


def kernel(hidden_states, weight):
    n_tokens = hidden_states.shape[0]
    blk = min(TOKEN_BLOCK, max(n_tokens // 2, 1))
    half = n_tokens // 2
    steps = half // blk

    la, lb, va, vb, ia, ib = pl.pallas_call(
        _router_block,
        grid=(steps,),
        in_specs=[
            pl.BlockSpec((blk, HIDDEN_DIM), lambda i: (i, 0)),
            pl.BlockSpec((blk, HIDDEN_DIM), lambda i, s=steps: (i + s, 0)),
            pl.BlockSpec((NUM_EXPERTS, HIDDEN_DIM), lambda i: (0, 0)),
        ],
        out_specs=[
            pl.BlockSpec((blk, NUM_EXPERTS), lambda i: (i, 0)),
            pl.BlockSpec((blk, NUM_EXPERTS), lambda i: (i, 0)),
            pl.BlockSpec((TOP_K, blk), lambda i: (0, i)),
            pl.BlockSpec((TOP_K, blk), lambda i: (0, i)),
            pl.BlockSpec((TOP_K, blk), lambda i: (0, i)),
            pl.BlockSpec((TOP_K, blk), lambda i: (0, i)),
        ],
        out_shape=[
            jax.ShapeDtypeStruct((half, NUM_EXPERTS), jnp.float32),
            jax.ShapeDtypeStruct((half, NUM_EXPERTS), jnp.float32),
            jax.ShapeDtypeStruct((TOP_K, half), jnp.float32),
            jax.ShapeDtypeStruct((TOP_K, half), jnp.float32),
            jax.ShapeDtypeStruct((TOP_K, half), jnp.int32),
            jax.ShapeDtypeStruct((TOP_K, half), jnp.int32),
        ],
        compiler_params=pltpu.CompilerParams(
            dimension_semantics=("arbitrary",),
        ),
    )(hidden_states, hidden_states, weight)
    logits = jnp.concatenate([la, lb], axis=0)
    topv = jnp.concatenate([va, vb], axis=1).T
    topi = jnp.concatenate([ia, ib], axis=1).T
    return (logits, topv, topi)
